# baseline (speedup 1.0000x reference)
import jax
import jax.numpy as jnp
from jax import lax
from jax.experimental import pallas as pl
from jax.experimental.pallas import tpu as pltpu
from jax.experimental.pallas import tpu_sc as plsc

NUM_FACTOR = 32
BATCH = 16384
NC = 2
NS = 16
L = 16
NW = NC * NS
B_PER_W = BATCH // NW


def _mf_body(user_hbm, item_hbm, p_hbm, q_hbm, bu_hbm, bi_hbm, out_hbm,
             uidx_v, iidx_v, prows_v, qrows_v, bu_v, bi_v, out_v,
             sem_p, sem_q, sem_bu, sem_bi):
    wid = lax.axis_index("s") * NC + lax.axis_index("c")
    base = wid * B_PER_W

    pltpu.sync_copy(user_hbm.at[pl.ds(base, B_PER_W)], uidx_v)
    pltpu.sync_copy(item_hbm.at[pl.ds(base, B_PER_W)], iidx_v)

    cp_p = pltpu.async_copy(p_hbm.at[uidx_v], prows_v, sem_p)
    cp_q = pltpu.async_copy(q_hbm.at[iidx_v], qrows_v, sem_q)
    cp_bu = pltpu.async_copy(bu_hbm.at[uidx_v], bu_v, sem_bu)
    cp_bi = pltpu.async_copy(bi_hbm.at[iidx_v], bi_v, sem_bi)
    cp_p.wait()
    cp_q.wait()
    cp_bu.wait()
    cp_bi.wait()

    lanes = lax.iota(jnp.int32, L)

    def group(g, _):
        rows = g * L + lanes
        accs = [jnp.zeros((L,), jnp.float32) for _ in range(4)]
        for f in range(NUM_FACTOR):
            cols = jnp.full((L,), f, jnp.int32)
            pv = plsc.load_gather(prows_v, [rows, cols])
            qv = plsc.load_gather(qrows_v, [rows, cols])
            accs[f % 4] = accs[f % 4] + pv * qv
        dot = (accs[0] + accs[1]) + (accs[2] + accs[3])
        out_v[pl.ds(g * L, L)] = dot + bu_v[pl.ds(g * L, L)] + bi_v[pl.ds(g * L, L)]
        return 0

    lax.fori_loop(0, B_PER_W // L, group, 0)

    pltpu.sync_copy(out_v, out_hbm.at[pl.ds(base, B_PER_W)])


@jax.jit
def _mf(user, item, p, q, b_u, b_i):
    mesh = plsc.VectorSubcoreMesh(
        core_axis_name="c", subcore_axis_name="s",
        num_cores=NC, num_subcores=NS)
    return pl.kernel(
        _mf_body,
        out_type=jax.ShapeDtypeStruct((BATCH,), jnp.float32),
        mesh=mesh,
        compiler_params=pltpu.CompilerParams(
            needs_layout_passes=False, use_tc_tiling_on_sc=False),
        scratch_types=[
            pltpu.VMEM((B_PER_W,), jnp.int32),
            pltpu.VMEM((B_PER_W,), jnp.int32),
            pltpu.VMEM((B_PER_W, NUM_FACTOR), jnp.float32),
            pltpu.VMEM((B_PER_W, NUM_FACTOR), jnp.float32),
            pltpu.VMEM((B_PER_W,), jnp.float32),
            pltpu.VMEM((B_PER_W,), jnp.float32),
            pltpu.VMEM((B_PER_W,), jnp.float32),
            pltpu.SemaphoreType.DMA,
            pltpu.SemaphoreType.DMA,
            pltpu.SemaphoreType.DMA,
            pltpu.SemaphoreType.DMA,
        ],
    )(user, item, p, q, b_u, b_i)


def kernel(user, item, p, q, b_u, b_i):
    return _mf(user, item, p, q, b_u, b_i)
